# baseline (speedup 1.0000x reference)
import functools

import jax
import jax.numpy as jnp
from jax import lax
from jax.experimental import pallas as pl
from jax.experimental.pallas import tpu as pltpu
from jax.experimental.pallas import tpu_sc as plsc

NC = 2
NS = 16
LANES = 16

W_CAT1 = 72
W_DST1 = 16
W_CAT2 = 48
W_DST2 = 16

_HIGH = jax.lax.Precision.HIGHEST


def _mm(a, b):
    return jnp.dot(a, b, precision=_HIGH, preferred_element_type=jnp.float32)


def _leaky_exp(e):
    return jnp.exp(jnp.maximum(e, 0.2 * e))


def _vgather(x, idx):
    return x.at[idx].get(mode="promise_in_bounds")


def _tc1_body(x_ref, w_ref, p_ref, d_ref, cat_ref, adst_ref):
    h = _mm(x_ref[...], w_ref[...])
    cat_ref[...] = _mm(h, p_ref[...])
    adst_ref[...] = _mm(h, d_ref[...])


def _tc2_body(a0_ref, a1_ref, cat_ref, adst_ref, b1_ref,
              s64_ref, sa_ref, sd_ref, e8_ref, wmu_ref, wstd_ref,
              pmu_ref, pstd_ref, qmu_ref, qstd_ref,
              cat2_ref, adst2_ref):
    accsum = a0_ref[0] + a1_ref[0]
    cat1 = cat_ref[...]
    h1 = _mm(cat1, s64_ref[...])
    asrc1 = _mm(cat1, sa_ref[...])
    num_e = _mm(accsum, s64_ref[...])
    s_e = _mm(accsum, sa_ref[...])
    ad1 = _mm(adst_ref[...], sd_ref[...])
    g_self = _leaky_exp(asrc1 + ad1)
    den64 = _mm(s_e + g_self, e8_ref[...])
    gexp64 = _mm(g_self, e8_ref[...])
    out1 = (num_e + gexp64 * h1) / (den64 + 1e-16) + b1_ref[...]
    h2 = jnp.maximum(out1, 0.0)
    hmu = _mm(h2, wmu_ref[...])
    hstd = _mm(h2, wstd_ref[...])
    cat2_ref[...] = _mm(hmu, pmu_ref[...]) + _mm(hstd, pstd_ref[...])
    adst2_ref[...] = _mm(hmu, qmu_ref[...]) + _mm(hstd, qstd_ref[...])


def _tc3_body(a0_ref, a1_ref, cat2_ref, adst2_ref, bmu_ref, bstd_ref,
              tmu_ref, tstd_ref, bsmu_ref, bsstd_ref, umu_ref, ustd_ref,
              mu_ref, std_ref):
    accsum = a0_ref[0] + a1_ref[0]
    cat2 = cat2_ref[...]
    adst2 = adst2_ref[...]
    hmu = _mm(cat2, tmu_ref[...])
    hstd = _mm(cat2, tstd_ref[...])
    amu_s = _mm(cat2, bsmu_ref[...])
    astd_s = _mm(cat2, bsstd_ref[...])
    admu = _mm(adst2, umu_ref[...])
    adstd = _mm(adst2, ustd_ref[...])
    gmu = _leaky_exp(amu_s + admu)
    gstd = _leaky_exp(astd_s + adstd)
    num_mu = _mm(accsum, tmu_ref[...])
    num_std = _mm(accsum, tstd_ref[...])
    s_mu = _mm(accsum, bsmu_ref[...])
    s_std = _mm(accsum, bsstd_ref[...])
    mu_ref[...] = (num_mu + gmu * hmu) / (s_mu + gmu + 1e-16) + bmu_ref[...]
    std_ref[...] = (num_std + gstd * hstd) / (s_std + gstd + 1e-16) + bstd_ref[...]


def _sc1_body(cat_hbm, adst_hbm, src_hbm, dst_hbm, out_hbm,
              idx_s, idx_d, rin0, rin1, dr0, dr1, rout0, rout1,
              acc, sg0, sg1, sd0, sd1, ss0, ss1,
              *, cpt, rows_per_tile, chunk_e):
    c = lax.axis_index("c")
    s = lax.axis_index("s")
    w = c * NS + s
    lane = lax.iota(jnp.int32, LANES)
    rowpat = lane >> 3
    colpat = lane & 7
    half = lane < 8
    zv = jnp.zeros((LANES,), jnp.float32)

    @plsc.parallel_loop(0, chunk_e, 1, unroll=4)
    def zrow(r):
        for q in range(W_CAT1 // LANES):
            rout0[r, pl.ds(q * LANES, LANES)] = zv
        rout0[r, pl.ds(W_CAT1 - LANES, LANES)] = zv
    base_r = s * rows_per_tile
    nfull, rem = divmod(rows_per_tile, chunk_e)
    for b in range(nfull):
        pltpu.sync_copy(rout0, acc.at[pl.ds(base_r + b * chunk_e, chunk_e)])
    if rem:
        pltpu.sync_copy(rout0.at[pl.ds(0, rem)],
                        acc.at[pl.ds(base_r + nfull * chunk_e, rem)])
    plsc.subcore_barrier()

    pltpu.sync_copy(src_hbm.at[pl.ds(w * cpt, cpt)], idx_s)
    pltpu.sync_copy(dst_hbm.at[pl.ds(w * cpt, cpt)], idx_d)

    bufs = ((rin0, dr0, rout0, sg0, sd0, ss0),
            (rin1, dr1, rout1, sg1, sd1, ss1))

    def gather(j, b):
        rin, dr, _, sg, sd, _ = bufs[b]
        pltpu.async_copy(cat_hbm.at[idx_s.at[j]], rin, sg)
        pltpu.async_copy(adst_hbm.at[idx_d.at[j]], dr, sd)

    def gather_wait(j, b):
        rin, dr, _, sg, sd, _ = bufs[b]
        pltpu.make_async_copy(cat_hbm.at[idx_s.at[j]], rin, sg).wait()
        pltpu.make_async_copy(adst_hbm.at[idx_d.at[j]], dr, sd).wait()

    def scatter(j, b):
        _, _, rout, _, _, ss = bufs[b]
        pltpu.async_copy(rout, acc.at[idx_d.at[j]], ss, add=True)

    def scatter_wait(j, b):
        _, _, rout, _, _, ss = bufs[b]
        pltpu.make_async_copy(rout, acc.at[idx_d.at[j]], ss).wait()

    def compute(b):
        rin, dr, rout, _, _, _ = bufs[b]

        @plsc.parallel_loop(0, chunk_e, 1, unroll=4)
        def sstep(i):
            av = rin[i, pl.ds(56, LANES)]
            dv = dr[i, pl.ds(0, LANES)]
            g = _leaky_exp(av + dv)
            v3s = av
            for k in range(4):
                gexp = _vgather(g, 8 + 2 * k + rowpat)
                vks = rin[i, pl.ds(16 * k, LANES)] * gexp
                rout[i, pl.ds(16 * k, LANES)] = vks
                if k == 3:
                    v3s = vks
            comb = jnp.where(half, _vgather(v3s, 8 + colpat),
                             _vgather(g, 8 + colpat))
            rout[i, pl.ds(56, LANES)] = comb

    gather(0, 0)

    def step(j, b):
        @pl.when(j + 1 < cpt)
        def _():
            gather(j + 1, 1 - b)
        gather_wait(j, b)

        @pl.when(j >= 2)
        def _():
            scatter_wait(j - 2, b)
        compute(b)
        scatter(j, b)

    def pair(p, _):
        step(2 * p, 0)
        step(2 * p + 1, 1)
        return 0
    lax.fori_loop(0, cpt // 2, pair, 0)
    scatter_wait(cpt - 2, 0)
    scatter_wait(cpt - 1, 1)

    plsc.subcore_barrier()
    pltpu.sync_copy(acc.at[pl.ds(base_r, rows_per_tile)],
                    out_hbm.at[c, pl.ds(base_r, rows_per_tile)])


def _sc2_body(cat_hbm, adst_hbm, src_hbm, dst_hbm, out_hbm,
              idx_s, idx_d, rin0, rin1, dr0, dr1, rout0, rout1,
              acc, sg0, sg1, sd0, sd1, ss0, ss1,
              *, cpt, rows_per_tile, chunk_e):
    c = lax.axis_index("c")
    s = lax.axis_index("s")
    w = c * NS + s
    lane = lax.iota(jnp.int32, LANES)
    ones_i = lane * 0 + 1
    zeros_i = lane * 0
    zv = jnp.zeros((LANES,), jnp.float32)

    @plsc.parallel_loop(0, chunk_e, 1, unroll=4)
    def zrow(r):
        for q in range(W_CAT2 // LANES):
            rout0[r, pl.ds(q * LANES, LANES)] = zv
    base_r = s * rows_per_tile
    nfull, rem = divmod(rows_per_tile, chunk_e)
    for b in range(nfull):
        pltpu.sync_copy(rout0, acc.at[pl.ds(base_r + b * chunk_e, chunk_e)])
    if rem:
        pltpu.sync_copy(rout0.at[pl.ds(0, rem)],
                        acc.at[pl.ds(base_r + nfull * chunk_e, rem)])
    plsc.subcore_barrier()

    pltpu.sync_copy(src_hbm.at[pl.ds(w * cpt, cpt)], idx_s)
    pltpu.sync_copy(dst_hbm.at[pl.ds(w * cpt, cpt)], idx_d)

    bufs = ((rin0, dr0, rout0, sg0, sd0, ss0),
            (rin1, dr1, rout1, sg1, sd1, ss1))

    def gather(j, b):
        rin, dr, _, sg, sd, _ = bufs[b]
        pltpu.async_copy(cat_hbm.at[idx_s.at[j]], rin, sg)
        pltpu.async_copy(adst_hbm.at[idx_d.at[j]], dr, sd)

    def gather_wait(j, b):
        rin, dr, _, sg, sd, _ = bufs[b]
        pltpu.make_async_copy(cat_hbm.at[idx_s.at[j]], rin, sg).wait()
        pltpu.make_async_copy(adst_hbm.at[idx_d.at[j]], dr, sd).wait()

    def scatter(j, b):
        _, _, rout, _, _, ss = bufs[b]
        pltpu.async_copy(rout, acc.at[idx_d.at[j]], ss, add=True)

    def scatter_wait(j, b):
        _, _, rout, _, _, ss = bufs[b]
        pltpu.make_async_copy(rout, acc.at[idx_d.at[j]], ss).wait()

    def compute(b):
        rin, dr, rout, _, _, _ = bufs[b]

        @plsc.parallel_loop(0, chunk_e, 1, unroll=4)
        def sstep(i):
            av = rin[i, pl.ds(0, LANES)]
            dv = dr[i, pl.ds(0, LANES)]
            g = _leaky_exp(av + dv)
            rout[i, pl.ds(0, LANES)] = g
            gmu = _vgather(g, zeros_i)
            rout[i, pl.ds(16, LANES)] = rin[i, pl.ds(16, LANES)] * gmu
            gstd = _vgather(g, ones_i)
            rout[i, pl.ds(32, LANES)] = rin[i, pl.ds(32, LANES)] * gstd

    gather(0, 0)

    def step(j, b):
        @pl.when(j + 1 < cpt)
        def _():
            gather(j + 1, 1 - b)
        gather_wait(j, b)

        @pl.when(j >= 2)
        def _():
            scatter_wait(j - 2, b)
        compute(b)
        scatter(j, b)

    def pair(p, _):
        step(2 * p, 0)
        step(2 * p + 1, 1)
        return 0
    lax.fori_loop(0, cpt // 2, pair, 0)
    scatter_wait(cpt - 2, 0)
    scatter_wait(cpt - 1, 1)

    plsc.subcore_barrier()
    pltpu.sync_copy(acc.at[pl.ds(base_r, rows_per_tile)],
                    out_hbm.at[c, pl.ds(base_r, rows_per_tile)])


def _expand_blockdiag(att, heads, ch):
    eye = jnp.eye(heads, dtype=jnp.float32)
    return (att[:, :, None] * eye[:, None, :]).reshape(heads * ch, heads)


def kernel(features, edges, W1, att_src1, att_dst1, b1,
           W_mu, att_src_mu, att_dst_mu, b_mu,
           W_std, att_src_std, att_dst_std, b_std):
    n, d_in = features.shape
    e_total = edges.shape[1]
    heads, hid = att_src1.shape[1], att_src1.shape[2]
    z = W_mu.shape[1]
    hh = heads * hid

    f32 = jnp.float32
    eye64 = jnp.eye(hh, dtype=f32)
    eye16 = jnp.eye(z, dtype=f32)

    asrc_bd = _expand_blockdiag(att_src1.reshape(heads, hid), heads, hid)
    adst_bd = _expand_blockdiag(att_dst1.reshape(heads, hid), heads, hid)
    P1 = jnp.concatenate([eye64, asrc_bd], axis=1)
    D1 = jnp.concatenate([jnp.zeros((hh, 8), f32), adst_bd], axis=1)

    S64 = jnp.concatenate([eye64, jnp.zeros((8, hh), f32)], axis=0)
    SA = jnp.concatenate([jnp.zeros((hh, heads), f32),
                          jnp.eye(heads, dtype=f32)], axis=0)
    E8 = jnp.repeat(jnp.eye(heads, dtype=f32), hid, axis=1)
    SD = jnp.concatenate([jnp.zeros((8, heads), f32),
                          jnp.eye(heads, dtype=f32)], axis=0)

    Pmu = jnp.zeros((z, W_CAT2), f32)
    Pmu = Pmu.at[:, 16:16 + z].set(eye16)
    Pmu = Pmu.at[:, 0].set(att_src_mu.reshape(z))
    Pstd = jnp.zeros((z, W_CAT2), f32)
    Pstd = Pstd.at[:, 16 + z:16 + 2 * z].set(eye16)
    Pstd = Pstd.at[:, 1].set(att_src_std.reshape(z))
    Qmu = jnp.zeros((z, W_DST2), f32).at[:, 0].set(att_dst_mu.reshape(z))
    Qstd = jnp.zeros((z, W_DST2), f32).at[:, 1].set(att_dst_std.reshape(z))

    Tmu = jnp.zeros((W_CAT2, z), f32).at[16:16 + z, :].set(eye16)
    Tstd = jnp.zeros((W_CAT2, z), f32).at[16 + z:16 + 2 * z, :].set(eye16)
    Bsmu = jnp.zeros((W_CAT2, z), f32).at[0, :].set(1.0)
    Bsstd = jnp.zeros((W_CAT2, z), f32).at[1, :].set(1.0)
    Umu = jnp.zeros((W_DST2, z), f32).at[0, :].set(1.0)
    Ustd = jnp.zeros((W_DST2, z), f32).at[1, :].set(1.0)

    chunk_e = 128
    rows_pad = -(-e_total // chunk_e)
    rows_pad = -(-rows_pad // (NC * NS * 8)) * (NC * NS * 8)
    e_pad = rows_pad * chunk_e
    cpt = rows_pad // (NC * NS)
    rpt = -(-(n + 1) // (NS * 8)) * 8
    n_pad = rpt * NS

    pad_idx = (n + jnp.arange(e_pad - e_total, dtype=edges.dtype)
               % jnp.asarray(n_pad - n, dtype=edges.dtype))
    src2d = jnp.concatenate([edges[0], pad_idx]).reshape(rows_pad, chunk_e)
    dst2d = jnp.concatenate([edges[1], pad_idx]).reshape(rows_pad, chunk_e)

    Rp = n_pad // 16
    gridp = (16,)
    R = 2000
    grid = (n // R,)

    def _blkp(w):
        return pl.BlockSpec((Rp, w), lambda i: (i, 0))

    def _blk(w):
        return pl.BlockSpec((R, w), lambda i: (i, 0))

    def _full(a):
        return pl.BlockSpec(a.shape, lambda i: tuple(0 for _ in a.shape))

    cat1p, adst1p = pl.pallas_call(
        _tc1_body,
        grid=gridp,
        in_specs=[_blkp(d_in), _full(W1), _full(P1), _full(D1)],
        out_specs=[_blkp(W_CAT1), _blkp(W_DST1)],
        out_shape=[jax.ShapeDtypeStruct((n_pad, W_CAT1), f32),
                   jax.ShapeDtypeStruct((n_pad, W_DST1), f32)],
    )(features, W1, P1, D1)

    mesh = plsc.VectorSubcoreMesh(core_axis_name="c", subcore_axis_name="s",
                                  num_cores=NC, num_subcores=NS)

    sc1 = pl.kernel(
        functools.partial(_sc1_body, cpt=cpt, rows_per_tile=rpt,
                          chunk_e=chunk_e),
        out_type=jax.ShapeDtypeStruct((NC, n_pad, W_CAT1), f32),
        mesh=mesh,
        compiler_params=pltpu.CompilerParams(use_tc_tiling_on_sc=False),
        scratch_types=[
            pltpu.VMEM((cpt, chunk_e), jnp.int32),
            pltpu.VMEM((cpt, chunk_e), jnp.int32),
            pltpu.VMEM((chunk_e, W_CAT1), f32),
            pltpu.VMEM((chunk_e, W_CAT1), f32),
            pltpu.VMEM((chunk_e, W_DST1), f32),
            pltpu.VMEM((chunk_e, W_DST1), f32),
            pltpu.VMEM((chunk_e, W_CAT1), f32),
            pltpu.VMEM((chunk_e, W_CAT1), f32),
            pltpu.VMEM_SHARED((n_pad, W_CAT1), f32),
            pltpu.SemaphoreType.DMA,
            pltpu.SemaphoreType.DMA,
            pltpu.SemaphoreType.DMA,
            pltpu.SemaphoreType.DMA,
            pltpu.SemaphoreType.DMA,
            pltpu.SemaphoreType.DMA,
        ],
    )
    acc1 = sc1(cat1p, adst1p, src2d, dst2d)

    b1r = b1.reshape(1, hh)
    cat2p, adst2p = pl.pallas_call(
        _tc2_body,
        grid=gridp,
        in_specs=[pl.BlockSpec((1, Rp, W_CAT1), lambda i: (0, i, 0)),
                  pl.BlockSpec((1, Rp, W_CAT1), lambda i: (1, i, 0)),
                  _blkp(W_CAT1), _blkp(W_DST1),
                  _full(b1r), _full(S64), _full(SA), _full(SD), _full(E8),
                  _full(W_mu), _full(W_std), _full(Pmu), _full(Pstd),
                  _full(Qmu), _full(Qstd)],
        out_specs=[_blkp(W_CAT2), _blkp(W_DST2)],
        out_shape=[jax.ShapeDtypeStruct((n_pad, W_CAT2), f32),
                   jax.ShapeDtypeStruct((n_pad, W_DST2), f32)],
    )(acc1, acc1, cat1p, adst1p, b1r,
      S64, SA, SD, E8, W_mu, W_std, Pmu, Pstd, Qmu, Qstd)

    sc2 = pl.kernel(
        functools.partial(_sc2_body, cpt=cpt, rows_per_tile=rpt,
                          chunk_e=chunk_e),
        out_type=jax.ShapeDtypeStruct((NC, n_pad, W_CAT2), f32),
        mesh=mesh,
        compiler_params=pltpu.CompilerParams(use_tc_tiling_on_sc=False),
        scratch_types=[
            pltpu.VMEM((cpt, chunk_e), jnp.int32),
            pltpu.VMEM((cpt, chunk_e), jnp.int32),
            pltpu.VMEM((chunk_e, W_CAT2), f32),
            pltpu.VMEM((chunk_e, W_CAT2), f32),
            pltpu.VMEM((chunk_e, W_DST2), f32),
            pltpu.VMEM((chunk_e, W_DST2), f32),
            pltpu.VMEM((chunk_e, W_CAT2), f32),
            pltpu.VMEM((chunk_e, W_CAT2), f32),
            pltpu.VMEM_SHARED((n_pad, W_CAT2), f32),
            pltpu.SemaphoreType.DMA,
            pltpu.SemaphoreType.DMA,
            pltpu.SemaphoreType.DMA,
            pltpu.SemaphoreType.DMA,
            pltpu.SemaphoreType.DMA,
            pltpu.SemaphoreType.DMA,
        ],
    )
    acc2 = sc2(cat2p, adst2p, src2d, dst2d)

    bmur = b_mu.reshape(1, z)
    bstdr = b_std.reshape(1, z)
    mu, std = pl.pallas_call(
        _tc3_body,
        grid=grid,
        in_specs=[pl.BlockSpec((1, R, W_CAT2), lambda i: (0, i, 0)),
                  pl.BlockSpec((1, R, W_CAT2), lambda i: (1, i, 0)),
                  _blk(W_CAT2), _blk(W_DST2),
                  _full(bmur), _full(bstdr), _full(Tmu), _full(Tstd),
                  _full(Bsmu), _full(Bsstd), _full(Umu), _full(Ustd)],
        out_specs=[_blk(z), _blk(z)],
        out_shape=[jax.ShapeDtypeStruct((n, z), f32),
                   jax.ShapeDtypeStruct((n, z), f32)],
    )(acc2, acc2, cat2p, adst2p, bmur, bstdr,
      Tmu, Tstd, Bsmu, Bsstd, Umu, Ustd)

    return (mu, std)


# speedup vs baseline: 1.0998x; 1.0998x over previous
import functools

import jax
import jax.numpy as jnp
from jax import lax
from jax.experimental import pallas as pl
from jax.experimental.pallas import tpu as pltpu
from jax.experimental.pallas import tpu_sc as plsc

NC = 2
NS = 16
LANES = 16

W_CAT1 = 72
W_DST1 = 16
W_CAT2 = 48
W_DST2 = 16

_HIGH = jax.lax.Precision.HIGHEST


def _mm(a, b):
    return jnp.dot(a, b, precision=_HIGH, preferred_element_type=jnp.float32)


def _leaky_exp(e):
    return jnp.exp(jnp.maximum(e, 0.2 * e))


def _vgather(x, idx):
    return x.at[idx].get(mode="promise_in_bounds")


def _tc1_body(x_ref, w_ref, p_ref, d_ref, cat_ref, adst_ref):
    h = _mm(x_ref[...], w_ref[...])
    cat_ref[...] = _mm(h, p_ref[...])
    adst_ref[...] = _mm(h, d_ref[...])


def _tc2_body(a0_ref, a1_ref, cat_ref, adst_ref, b1_ref,
              s64_ref, sa_ref, sd_ref, e8_ref, wmu_ref, wstd_ref,
              pmu_ref, pstd_ref, qmu_ref, qstd_ref,
              cat2_ref, adst2_ref):
    accsum = a0_ref[...] + a1_ref[...]
    cat1 = cat_ref[...]
    h1 = _mm(cat1, s64_ref[...])
    asrc1 = _mm(cat1, sa_ref[...])
    num_e = _mm(accsum, s64_ref[...])
    s_e = _mm(accsum, sa_ref[...])
    ad1 = _mm(adst_ref[...], sd_ref[...])
    g_self = _leaky_exp(asrc1 + ad1)
    den64 = _mm(s_e + g_self, e8_ref[...])
    gexp64 = _mm(g_self, e8_ref[...])
    out1 = (num_e + gexp64 * h1) / (den64 + 1e-16) + b1_ref[...]
    h2 = jnp.maximum(out1, 0.0)
    hmu = _mm(h2, wmu_ref[...])
    hstd = _mm(h2, wstd_ref[...])
    cat2_ref[...] = _mm(hmu, pmu_ref[...]) + _mm(hstd, pstd_ref[...])
    adst2_ref[...] = _mm(hmu, qmu_ref[...]) + _mm(hstd, qstd_ref[...])


def _tc3_body(a0_ref, a1_ref, cat2_ref, adst2_ref, bmu_ref, bstd_ref,
              tmu_ref, tstd_ref, bsmu_ref, bsstd_ref, umu_ref, ustd_ref,
              mu_ref, std_ref):
    accsum = a0_ref[...] + a1_ref[...]
    cat2 = cat2_ref[...]
    adst2 = adst2_ref[...]
    hmu = _mm(cat2, tmu_ref[...])
    hstd = _mm(cat2, tstd_ref[...])
    amu_s = _mm(cat2, bsmu_ref[...])
    astd_s = _mm(cat2, bsstd_ref[...])
    admu = _mm(adst2, umu_ref[...])
    adstd = _mm(adst2, ustd_ref[...])
    gmu = _leaky_exp(amu_s + admu)
    gstd = _leaky_exp(astd_s + adstd)
    num_mu = _mm(accsum, tmu_ref[...])
    num_std = _mm(accsum, tstd_ref[...])
    s_mu = _mm(accsum, bsmu_ref[...])
    s_std = _mm(accsum, bsstd_ref[...])
    mu_ref[...] = (num_mu + gmu * hmu) / (s_mu + gmu + 1e-16) + bmu_ref[...]
    std_ref[...] = (num_std + gstd * hstd) / (s_std + gstd + 1e-16) + bstd_ref[...]


def _sc1_body(cat_hbm, adst_hbm, src_hbm, dst_hbm, out_hbm,
              idx_s, idx_d, rin0, rin1, dr0, dr1, rout0, rout1,
              acc, sg0, sg1, sd0, sd1, ss0, ss1,
              *, cpt, rows_per_tile, chunk_e):
    c = lax.axis_index("c")
    s = lax.axis_index("s")
    w = c * NS + s
    lane = lax.iota(jnp.int32, LANES)
    rowpat = lane >> 3
    colpat = lane & 7
    half = lane < 8
    zv = jnp.zeros((LANES,), jnp.float32)

    @plsc.parallel_loop(0, chunk_e, 1, unroll=4)
    def zrow(r):
        for q in range(W_CAT1 // LANES):
            rout0[r, pl.ds(q * LANES, LANES)] = zv
        rout0[r, pl.ds(W_CAT1 - LANES, LANES)] = zv
    base_r = s * rows_per_tile
    nfull, rem = divmod(rows_per_tile, chunk_e)
    for b in range(nfull):
        pltpu.sync_copy(rout0, acc.at[pl.ds(base_r + b * chunk_e, chunk_e)])
    if rem:
        pltpu.sync_copy(rout0.at[pl.ds(0, rem)],
                        acc.at[pl.ds(base_r + nfull * chunk_e, rem)])
    plsc.subcore_barrier()

    pltpu.sync_copy(src_hbm.at[pl.ds(w * cpt, cpt)], idx_s)
    pltpu.sync_copy(dst_hbm.at[pl.ds(w * cpt, cpt)], idx_d)

    bufs = ((rin0, dr0, rout0, sg0, sd0, ss0),
            (rin1, dr1, rout1, sg1, sd1, ss1))

    def gather(j, b):
        rin, dr, _, sg, sd, _ = bufs[b]
        pltpu.async_copy(cat_hbm.at[idx_s.at[j]], rin, sg)
        pltpu.async_copy(adst_hbm.at[idx_d.at[j]], dr, sd)

    def gather_wait(j, b):
        rin, dr, _, sg, sd, _ = bufs[b]
        pltpu.make_async_copy(cat_hbm.at[idx_s.at[j]], rin, sg).wait()
        pltpu.make_async_copy(adst_hbm.at[idx_d.at[j]], dr, sd).wait()

    def scatter(j, b):
        _, _, rout, _, _, ss = bufs[b]
        pltpu.async_copy(rout, acc.at[idx_d.at[j]], ss, add=True)

    def scatter_wait(j, b):
        _, _, rout, _, _, ss = bufs[b]
        pltpu.make_async_copy(rout, acc.at[idx_d.at[j]], ss).wait()

    def compute(b):
        rin, dr, rout, _, _, _ = bufs[b]

        @plsc.parallel_loop(0, chunk_e, 1, unroll=4)
        def sstep(i):
            av = rin[i, pl.ds(56, LANES)]
            dv = dr[i, pl.ds(0, LANES)]
            g = _leaky_exp(av + dv)
            v3s = av
            for k in range(4):
                gexp = _vgather(g, 8 + 2 * k + rowpat)
                vks = rin[i, pl.ds(16 * k, LANES)] * gexp
                rout[i, pl.ds(16 * k, LANES)] = vks
                if k == 3:
                    v3s = vks
            comb = jnp.where(half, _vgather(v3s, 8 + colpat),
                             _vgather(g, 8 + colpat))
            rout[i, pl.ds(56, LANES)] = comb

    gather(0, 0)

    def step(j, b):
        @pl.when(j + 1 < cpt)
        def _():
            gather(j + 1, 1 - b)
        gather_wait(j, b)

        @pl.when(j >= 2)
        def _():
            scatter_wait(j - 2, b)
        compute(b)
        scatter(j, b)

    def pair(p, _):
        step(2 * p, 0)
        step(2 * p + 1, 1)
        return 0
    lax.fori_loop(0, cpt // 2, pair, 0)
    scatter_wait(cpt - 2, 0)
    scatter_wait(cpt - 1, 1)

    plsc.subcore_barrier()
    pltpu.sync_copy(acc.at[pl.ds(base_r, rows_per_tile)],
                    out_hbm.at[c, pl.ds(base_r, rows_per_tile)])


def _sc2_body(cat_hbm, adst_hbm, src_hbm, dst_hbm, out_hbm,
              idx_s, idx_d, rin0, rin1, dr0, dr1, rout0, rout1,
              acc, sg0, sg1, sd0, sd1, ss0, ss1,
              *, cpt, rows_per_tile, chunk_e):
    c = lax.axis_index("c")
    s = lax.axis_index("s")
    w = c * NS + s
    lane = lax.iota(jnp.int32, LANES)
    ones_i = lane * 0 + 1
    zeros_i = lane * 0
    zv = jnp.zeros((LANES,), jnp.float32)

    @plsc.parallel_loop(0, chunk_e, 1, unroll=4)
    def zrow(r):
        for q in range(W_CAT2 // LANES):
            rout0[r, pl.ds(q * LANES, LANES)] = zv
    base_r = s * rows_per_tile
    nfull, rem = divmod(rows_per_tile, chunk_e)
    for b in range(nfull):
        pltpu.sync_copy(rout0, acc.at[pl.ds(base_r + b * chunk_e, chunk_e)])
    if rem:
        pltpu.sync_copy(rout0.at[pl.ds(0, rem)],
                        acc.at[pl.ds(base_r + nfull * chunk_e, rem)])
    plsc.subcore_barrier()

    pltpu.sync_copy(src_hbm.at[pl.ds(w * cpt, cpt)], idx_s)
    pltpu.sync_copy(dst_hbm.at[pl.ds(w * cpt, cpt)], idx_d)

    bufs = ((rin0, dr0, rout0, sg0, sd0, ss0),
            (rin1, dr1, rout1, sg1, sd1, ss1))

    def gather(j, b):
        rin, dr, _, sg, sd, _ = bufs[b]
        pltpu.async_copy(cat_hbm.at[idx_s.at[j]], rin, sg)
        pltpu.async_copy(adst_hbm.at[idx_d.at[j]], dr, sd)

    def gather_wait(j, b):
        rin, dr, _, sg, sd, _ = bufs[b]
        pltpu.make_async_copy(cat_hbm.at[idx_s.at[j]], rin, sg).wait()
        pltpu.make_async_copy(adst_hbm.at[idx_d.at[j]], dr, sd).wait()

    def scatter(j, b):
        _, _, rout, _, _, ss = bufs[b]
        pltpu.async_copy(rout, acc.at[idx_d.at[j]], ss, add=True)

    def scatter_wait(j, b):
        _, _, rout, _, _, ss = bufs[b]
        pltpu.make_async_copy(rout, acc.at[idx_d.at[j]], ss).wait()

    def compute(b):
        rin, dr, rout, _, _, _ = bufs[b]

        @plsc.parallel_loop(0, chunk_e, 1, unroll=4)
        def sstep(i):
            av = rin[i, pl.ds(0, LANES)]
            dv = dr[i, pl.ds(0, LANES)]
            g = _leaky_exp(av + dv)
            rout[i, pl.ds(0, LANES)] = g
            gmu = _vgather(g, zeros_i)
            rout[i, pl.ds(16, LANES)] = rin[i, pl.ds(16, LANES)] * gmu
            gstd = _vgather(g, ones_i)
            rout[i, pl.ds(32, LANES)] = rin[i, pl.ds(32, LANES)] * gstd

    gather(0, 0)

    def step(j, b):
        @pl.when(j + 1 < cpt)
        def _():
            gather(j + 1, 1 - b)
        gather_wait(j, b)

        @pl.when(j >= 2)
        def _():
            scatter_wait(j - 2, b)
        compute(b)
        scatter(j, b)

    def pair(p, _):
        step(2 * p, 0)
        step(2 * p + 1, 1)
        return 0
    lax.fori_loop(0, cpt // 2, pair, 0)
    scatter_wait(cpt - 2, 0)
    scatter_wait(cpt - 1, 1)

    plsc.subcore_barrier()
    pltpu.sync_copy(acc.at[pl.ds(base_r, rows_per_tile)],
                    out_hbm.at[c, pl.ds(base_r, rows_per_tile)])


def _expand_blockdiag(att, heads, ch):
    eye = jnp.eye(heads, dtype=jnp.float32)
    return (att[:, :, None] * eye[:, None, :]).reshape(heads * ch, heads)


def kernel(features, edges, W1, att_src1, att_dst1, b1,
           W_mu, att_src_mu, att_dst_mu, b_mu,
           W_std, att_src_std, att_dst_std, b_std):
    n, d_in = features.shape
    e_total = edges.shape[1]
    heads, hid = att_src1.shape[1], att_src1.shape[2]
    z = W_mu.shape[1]
    hh = heads * hid

    f32 = jnp.float32
    eye64 = jnp.eye(hh, dtype=f32)
    eye16 = jnp.eye(z, dtype=f32)

    asrc_bd = _expand_blockdiag(att_src1.reshape(heads, hid), heads, hid)
    adst_bd = _expand_blockdiag(att_dst1.reshape(heads, hid), heads, hid)
    P1 = jnp.concatenate([eye64, asrc_bd], axis=1)
    D1 = jnp.concatenate([jnp.zeros((hh, 8), f32), adst_bd], axis=1)

    S64 = jnp.concatenate([eye64, jnp.zeros((8, hh), f32)], axis=0)
    SA = jnp.concatenate([jnp.zeros((hh, heads), f32),
                          jnp.eye(heads, dtype=f32)], axis=0)
    E8 = jnp.repeat(jnp.eye(heads, dtype=f32), hid, axis=1)
    SD = jnp.concatenate([jnp.zeros((8, heads), f32),
                          jnp.eye(heads, dtype=f32)], axis=0)

    Pmu = jnp.zeros((z, W_CAT2), f32)
    Pmu = Pmu.at[:, 16:16 + z].set(eye16)
    Pmu = Pmu.at[:, 0].set(att_src_mu.reshape(z))
    Pstd = jnp.zeros((z, W_CAT2), f32)
    Pstd = Pstd.at[:, 16 + z:16 + 2 * z].set(eye16)
    Pstd = Pstd.at[:, 1].set(att_src_std.reshape(z))
    Qmu = jnp.zeros((z, W_DST2), f32).at[:, 0].set(att_dst_mu.reshape(z))
    Qstd = jnp.zeros((z, W_DST2), f32).at[:, 1].set(att_dst_std.reshape(z))

    Tmu = jnp.zeros((W_CAT2, z), f32).at[16:16 + z, :].set(eye16)
    Tstd = jnp.zeros((W_CAT2, z), f32).at[16 + z:16 + 2 * z, :].set(eye16)
    Bsmu = jnp.zeros((W_CAT2, z), f32).at[0, :].set(1.0)
    Bsstd = jnp.zeros((W_CAT2, z), f32).at[1, :].set(1.0)
    Umu = jnp.zeros((W_DST2, z), f32).at[0, :].set(1.0)
    Ustd = jnp.zeros((W_DST2, z), f32).at[1, :].set(1.0)

    chunk_e = 128
    rows_pad = -(-e_total // chunk_e)
    rows_pad = -(-rows_pad // (NC * NS * 8)) * (NC * NS * 8)
    e_pad = rows_pad * chunk_e
    cpt = rows_pad // (NC * NS)
    rpt = -(-(n + 1) // (NS * 8)) * 8
    n_pad = rpt * NS

    pad_idx = (n + jnp.arange(e_pad - e_total, dtype=edges.dtype)
               % jnp.asarray(n_pad - n, dtype=edges.dtype))
    src2d = jnp.concatenate([edges[0], pad_idx]).reshape(rows_pad, chunk_e)
    dst2d = jnp.concatenate([edges[1], pad_idx]).reshape(rows_pad, chunk_e)

    R = 2000
    grid = (n // R,)

    def _blk(w):
        return pl.BlockSpec((R, w), lambda i: (i, 0))

    def _full(a):
        return pl.BlockSpec(a.shape, lambda i: tuple(0 for _ in a.shape))

    cat1, adst1 = pl.pallas_call(
        _tc1_body,
        grid=grid,
        in_specs=[_blk(d_in), _full(W1), _full(P1), _full(D1)],
        out_specs=[_blk(W_CAT1), _blk(W_DST1)],
        out_shape=[jax.ShapeDtypeStruct((n, W_CAT1), f32),
                   jax.ShapeDtypeStruct((n, W_DST1), f32)],
    )(features, W1, P1, D1)
    cat1p = jnp.pad(cat1, ((0, n_pad - n), (0, 0)))
    adst1p = jnp.pad(adst1, ((0, n_pad - n), (0, 0)))

    mesh = plsc.VectorSubcoreMesh(core_axis_name="c", subcore_axis_name="s",
                                  num_cores=NC, num_subcores=NS)

    sc1 = pl.kernel(
        functools.partial(_sc1_body, cpt=cpt, rows_per_tile=rpt,
                          chunk_e=chunk_e),
        out_type=jax.ShapeDtypeStruct((NC, n_pad, W_CAT1), f32),
        mesh=mesh,
        compiler_params=pltpu.CompilerParams(use_tc_tiling_on_sc=False),
        scratch_types=[
            pltpu.VMEM((cpt, chunk_e), jnp.int32),
            pltpu.VMEM((cpt, chunk_e), jnp.int32),
            pltpu.VMEM((chunk_e, W_CAT1), f32),
            pltpu.VMEM((chunk_e, W_CAT1), f32),
            pltpu.VMEM((chunk_e, W_DST1), f32),
            pltpu.VMEM((chunk_e, W_DST1), f32),
            pltpu.VMEM((chunk_e, W_CAT1), f32),
            pltpu.VMEM((chunk_e, W_CAT1), f32),
            pltpu.VMEM_SHARED((n_pad, W_CAT1), f32),
            pltpu.SemaphoreType.DMA,
            pltpu.SemaphoreType.DMA,
            pltpu.SemaphoreType.DMA,
            pltpu.SemaphoreType.DMA,
            pltpu.SemaphoreType.DMA,
            pltpu.SemaphoreType.DMA,
        ],
    )
    acc1 = sc1(cat1p, adst1p, src2d, dst2d)

    b1r = b1.reshape(1, hh)
    cat2, adst2 = pl.pallas_call(
        _tc2_body,
        grid=grid,
        in_specs=[_blk(W_CAT1), _blk(W_CAT1), _blk(W_CAT1), _blk(W_DST1),
                  _full(b1r), _full(S64), _full(SA), _full(SD), _full(E8),
                  _full(W_mu), _full(W_std), _full(Pmu), _full(Pstd),
                  _full(Qmu), _full(Qstd)],
        out_specs=[_blk(W_CAT2), _blk(W_DST2)],
        out_shape=[jax.ShapeDtypeStruct((n, W_CAT2), f32),
                   jax.ShapeDtypeStruct((n, W_DST2), f32)],
    )(acc1[0, :n], acc1[1, :n], cat1, adst1, b1r,
      S64, SA, SD, E8, W_mu, W_std, Pmu, Pstd, Qmu, Qstd)

    cat2p = jnp.pad(cat2, ((0, n_pad - n), (0, 0)))
    adst2p = jnp.pad(adst2, ((0, n_pad - n), (0, 0)))
    sc2 = pl.kernel(
        functools.partial(_sc2_body, cpt=cpt, rows_per_tile=rpt,
                          chunk_e=chunk_e),
        out_type=jax.ShapeDtypeStruct((NC, n_pad, W_CAT2), f32),
        mesh=mesh,
        compiler_params=pltpu.CompilerParams(use_tc_tiling_on_sc=False),
        scratch_types=[
            pltpu.VMEM((cpt, chunk_e), jnp.int32),
            pltpu.VMEM((cpt, chunk_e), jnp.int32),
            pltpu.VMEM((chunk_e, W_CAT2), f32),
            pltpu.VMEM((chunk_e, W_CAT2), f32),
            pltpu.VMEM((chunk_e, W_DST2), f32),
            pltpu.VMEM((chunk_e, W_DST2), f32),
            pltpu.VMEM((chunk_e, W_CAT2), f32),
            pltpu.VMEM((chunk_e, W_CAT2), f32),
            pltpu.VMEM_SHARED((n_pad, W_CAT2), f32),
            pltpu.SemaphoreType.DMA,
            pltpu.SemaphoreType.DMA,
            pltpu.SemaphoreType.DMA,
            pltpu.SemaphoreType.DMA,
            pltpu.SemaphoreType.DMA,
            pltpu.SemaphoreType.DMA,
        ],
    )
    acc2 = sc2(cat2p, adst2p, src2d, dst2d)

    bmur = b_mu.reshape(1, z)
    bstdr = b_std.reshape(1, z)
    mu, std = pl.pallas_call(
        _tc3_body,
        grid=grid,
        in_specs=[_blk(W_CAT2), _blk(W_CAT2), _blk(W_CAT2), _blk(W_DST2),
                  _full(bmur), _full(bstdr), _full(Tmu), _full(Tstd),
                  _full(Bsmu), _full(Bsstd), _full(Umu), _full(Ustd)],
        out_specs=[_blk(z), _blk(z)],
        out_shape=[jax.ShapeDtypeStruct((n, z), f32),
                   jax.ShapeDtypeStruct((n, z), f32)],
    )(acc2[0, :n], acc2[1, :n], cat2, adst2, bmur, bstdr,
      Tmu, Tstd, Bsmu, Bsstd, Umu, Ustd)

    return (mu, std)


# speedup vs baseline: 1.1017x; 1.0017x over previous
import functools

import jax
import jax.numpy as jnp
from jax import lax
from jax.experimental import pallas as pl
from jax.experimental.pallas import tpu as pltpu
from jax.experimental.pallas import tpu_sc as plsc

NC = 2
NS = 16
LANES = 16

W_CAT1 = 72
W_DST1 = 16
W_CAT2 = 48
W_DST2 = 16

_HIGH = jax.lax.Precision.HIGHEST


def _mm(a, b):
    return jnp.dot(a, b, precision=_HIGH, preferred_element_type=jnp.float32)


def _leaky_exp(e):
    return jnp.exp(jnp.maximum(e, 0.2 * e))


def _vgather(x, idx):
    return x.at[idx].get(mode="promise_in_bounds")


def _tc1_body(x_ref, w_ref, p_ref, d_ref, cat_ref, adst_ref):
    h = _mm(x_ref[...], w_ref[...])
    cat_ref[...] = _mm(h, p_ref[...])
    adst_ref[...] = _mm(h, d_ref[...])


def _tc2_body(a0_ref, a1_ref, cat_ref, adst_ref, b1_ref,
              s64_ref, sa_ref, sd_ref, e8_ref, wmu_ref, wstd_ref,
              pmu_ref, pstd_ref, qmu_ref, qstd_ref,
              cat2_ref, adst2_ref):
    accsum = a0_ref[...] + a1_ref[...]
    cat1 = cat_ref[...]
    h1 = _mm(cat1, s64_ref[...])
    asrc1 = _mm(cat1, sa_ref[...])
    num_e = _mm(accsum, s64_ref[...])
    s_e = _mm(accsum, sa_ref[...])
    ad1 = _mm(adst_ref[...], sd_ref[...])
    g_self = _leaky_exp(asrc1 + ad1)
    den64 = _mm(s_e + g_self, e8_ref[...])
    gexp64 = _mm(g_self, e8_ref[...])
    out1 = (num_e + gexp64 * h1) / (den64 + 1e-16) + b1_ref[...]
    h2 = jnp.maximum(out1, 0.0)
    hmu = _mm(h2, wmu_ref[...])
    hstd = _mm(h2, wstd_ref[...])
    cat2_ref[...] = _mm(hmu, pmu_ref[...]) + _mm(hstd, pstd_ref[...])
    adst2_ref[...] = _mm(hmu, qmu_ref[...]) + _mm(hstd, qstd_ref[...])


def _tc3_body(a0_ref, a1_ref, cat2_ref, adst2_ref, bmu_ref, bstd_ref,
              tmu_ref, tstd_ref, bsmu_ref, bsstd_ref, umu_ref, ustd_ref,
              mu_ref, std_ref):
    accsum = a0_ref[...] + a1_ref[...]
    cat2 = cat2_ref[...]
    adst2 = adst2_ref[...]
    hmu = _mm(cat2, tmu_ref[...])
    hstd = _mm(cat2, tstd_ref[...])
    amu_s = _mm(cat2, bsmu_ref[...])
    astd_s = _mm(cat2, bsstd_ref[...])
    admu = _mm(adst2, umu_ref[...])
    adstd = _mm(adst2, ustd_ref[...])
    gmu = _leaky_exp(amu_s + admu)
    gstd = _leaky_exp(astd_s + adstd)
    num_mu = _mm(accsum, tmu_ref[...])
    num_std = _mm(accsum, tstd_ref[...])
    s_mu = _mm(accsum, bsmu_ref[...])
    s_std = _mm(accsum, bsstd_ref[...])
    mu_ref[...] = (num_mu + gmu * hmu) / (s_mu + gmu + 1e-16) + bmu_ref[...]
    std_ref[...] = (num_std + gstd * hstd) / (s_std + gstd + 1e-16) + bstd_ref[...]


def _sc1_body(cat_hbm, adst_hbm, src_hbm, dst_hbm, out_hbm,
              idx_s, idx_d, rin0, rin1, dr0, dr1, rout0, rout1,
              acc, sg0, sg1, sd0, sd1, ss0, ss1,
              *, cpt, rows_per_tile, chunk_e):
    c = lax.axis_index("c")
    s = lax.axis_index("s")
    w = c * NS + s
    lane = lax.iota(jnp.int32, LANES)
    rowpat = lane >> 3
    colpat = lane & 7
    half = lane < 8
    zv = jnp.zeros((LANES,), jnp.float32)

    @plsc.parallel_loop(0, chunk_e, 1, unroll=8)
    def zrow(r):
        for q in range(W_CAT1 // LANES):
            rout0[r, pl.ds(q * LANES, LANES)] = zv
        rout0[r, pl.ds(W_CAT1 - LANES, LANES)] = zv
    base_r = s * rows_per_tile
    nfull, rem = divmod(rows_per_tile, chunk_e)
    for b in range(nfull):
        pltpu.sync_copy(rout0, acc.at[pl.ds(base_r + b * chunk_e, chunk_e)])
    if rem:
        pltpu.sync_copy(rout0.at[pl.ds(0, rem)],
                        acc.at[pl.ds(base_r + nfull * chunk_e, rem)])
    plsc.subcore_barrier()

    pltpu.sync_copy(src_hbm.at[pl.ds(w * cpt, cpt)], idx_s)
    pltpu.sync_copy(dst_hbm.at[pl.ds(w * cpt, cpt)], idx_d)

    bufs = ((rin0, dr0, rout0, sg0, sd0, ss0),
            (rin1, dr1, rout1, sg1, sd1, ss1))

    def gather(j, b):
        rin, dr, _, sg, sd, _ = bufs[b]
        pltpu.async_copy(cat_hbm.at[idx_s.at[j]], rin, sg)
        pltpu.async_copy(adst_hbm.at[idx_d.at[j]], dr, sd)

    def gather_wait(j, b):
        rin, dr, _, sg, sd, _ = bufs[b]
        pltpu.make_async_copy(cat_hbm.at[idx_s.at[j]], rin, sg).wait()
        pltpu.make_async_copy(adst_hbm.at[idx_d.at[j]], dr, sd).wait()

    def scatter(j, b):
        _, _, rout, _, _, ss = bufs[b]
        pltpu.async_copy(rout, acc.at[idx_d.at[j]], ss, add=True)

    def scatter_wait(j, b):
        _, _, rout, _, _, ss = bufs[b]
        pltpu.make_async_copy(rout, acc.at[idx_d.at[j]], ss).wait()

    def compute(b):
        rin, dr, rout, _, _, _ = bufs[b]

        @plsc.parallel_loop(0, chunk_e, 1, unroll=8)
        def sstep(i):
            av = rin[i, pl.ds(56, LANES)]
            dv = dr[i, pl.ds(0, LANES)]
            g = _leaky_exp(av + dv)
            v3s = av
            for k in range(4):
                gexp = _vgather(g, 8 + 2 * k + rowpat)
                vks = rin[i, pl.ds(16 * k, LANES)] * gexp
                rout[i, pl.ds(16 * k, LANES)] = vks
                if k == 3:
                    v3s = vks
            comb = jnp.where(half, _vgather(v3s, 8 + colpat),
                             _vgather(g, 8 + colpat))
            rout[i, pl.ds(56, LANES)] = comb

    gather(0, 0)

    def step(j, b):
        @pl.when(j + 1 < cpt)
        def _():
            gather(j + 1, 1 - b)
        gather_wait(j, b)

        @pl.when(j >= 2)
        def _():
            scatter_wait(j - 2, b)
        compute(b)
        scatter(j, b)

    def pair(p, _):
        step(2 * p, 0)
        step(2 * p + 1, 1)
        return 0
    lax.fori_loop(0, cpt // 2, pair, 0)
    scatter_wait(cpt - 2, 0)
    scatter_wait(cpt - 1, 1)

    plsc.subcore_barrier()
    pltpu.sync_copy(acc.at[pl.ds(base_r, rows_per_tile)],
                    out_hbm.at[c, pl.ds(base_r, rows_per_tile)])


def _sc2_body(cat_hbm, adst_hbm, src_hbm, dst_hbm, out_hbm,
              idx_s, idx_d, rin0, rin1, dr0, dr1, rout0, rout1,
              acc, sg0, sg1, sd0, sd1, ss0, ss1,
              *, cpt, rows_per_tile, chunk_e):
    c = lax.axis_index("c")
    s = lax.axis_index("s")
    w = c * NS + s
    lane = lax.iota(jnp.int32, LANES)
    ones_i = lane * 0 + 1
    zeros_i = lane * 0
    zv = jnp.zeros((LANES,), jnp.float32)

    @plsc.parallel_loop(0, chunk_e, 1, unroll=8)
    def zrow(r):
        for q in range(W_CAT2 // LANES):
            rout0[r, pl.ds(q * LANES, LANES)] = zv
    base_r = s * rows_per_tile
    nfull, rem = divmod(rows_per_tile, chunk_e)
    for b in range(nfull):
        pltpu.sync_copy(rout0, acc.at[pl.ds(base_r + b * chunk_e, chunk_e)])
    if rem:
        pltpu.sync_copy(rout0.at[pl.ds(0, rem)],
                        acc.at[pl.ds(base_r + nfull * chunk_e, rem)])
    plsc.subcore_barrier()

    pltpu.sync_copy(src_hbm.at[pl.ds(w * cpt, cpt)], idx_s)
    pltpu.sync_copy(dst_hbm.at[pl.ds(w * cpt, cpt)], idx_d)

    bufs = ((rin0, dr0, rout0, sg0, sd0, ss0),
            (rin1, dr1, rout1, sg1, sd1, ss1))

    def gather(j, b):
        rin, dr, _, sg, sd, _ = bufs[b]
        pltpu.async_copy(cat_hbm.at[idx_s.at[j]], rin, sg)
        pltpu.async_copy(adst_hbm.at[idx_d.at[j]], dr, sd)

    def gather_wait(j, b):
        rin, dr, _, sg, sd, _ = bufs[b]
        pltpu.make_async_copy(cat_hbm.at[idx_s.at[j]], rin, sg).wait()
        pltpu.make_async_copy(adst_hbm.at[idx_d.at[j]], dr, sd).wait()

    def scatter(j, b):
        _, _, rout, _, _, ss = bufs[b]
        pltpu.async_copy(rout, acc.at[idx_d.at[j]], ss, add=True)

    def scatter_wait(j, b):
        _, _, rout, _, _, ss = bufs[b]
        pltpu.make_async_copy(rout, acc.at[idx_d.at[j]], ss).wait()

    def compute(b):
        rin, dr, rout, _, _, _ = bufs[b]

        @plsc.parallel_loop(0, chunk_e, 1, unroll=8)
        def sstep(i):
            av = rin[i, pl.ds(0, LANES)]
            dv = dr[i, pl.ds(0, LANES)]
            g = _leaky_exp(av + dv)
            rout[i, pl.ds(0, LANES)] = g
            gmu = _vgather(g, zeros_i)
            rout[i, pl.ds(16, LANES)] = rin[i, pl.ds(16, LANES)] * gmu
            gstd = _vgather(g, ones_i)
            rout[i, pl.ds(32, LANES)] = rin[i, pl.ds(32, LANES)] * gstd

    gather(0, 0)

    def step(j, b):
        @pl.when(j + 1 < cpt)
        def _():
            gather(j + 1, 1 - b)
        gather_wait(j, b)

        @pl.when(j >= 2)
        def _():
            scatter_wait(j - 2, b)
        compute(b)
        scatter(j, b)

    def pair(p, _):
        step(2 * p, 0)
        step(2 * p + 1, 1)
        return 0
    lax.fori_loop(0, cpt // 2, pair, 0)
    scatter_wait(cpt - 2, 0)
    scatter_wait(cpt - 1, 1)

    plsc.subcore_barrier()
    pltpu.sync_copy(acc.at[pl.ds(base_r, rows_per_tile)],
                    out_hbm.at[c, pl.ds(base_r, rows_per_tile)])


def _expand_blockdiag(att, heads, ch):
    eye = jnp.eye(heads, dtype=jnp.float32)
    return (att[:, :, None] * eye[:, None, :]).reshape(heads * ch, heads)


def kernel(features, edges, W1, att_src1, att_dst1, b1,
           W_mu, att_src_mu, att_dst_mu, b_mu,
           W_std, att_src_std, att_dst_std, b_std):
    n, d_in = features.shape
    e_total = edges.shape[1]
    heads, hid = att_src1.shape[1], att_src1.shape[2]
    z = W_mu.shape[1]
    hh = heads * hid

    f32 = jnp.float32
    eye64 = jnp.eye(hh, dtype=f32)
    eye16 = jnp.eye(z, dtype=f32)

    asrc_bd = _expand_blockdiag(att_src1.reshape(heads, hid), heads, hid)
    adst_bd = _expand_blockdiag(att_dst1.reshape(heads, hid), heads, hid)
    P1 = jnp.concatenate([eye64, asrc_bd], axis=1)
    D1 = jnp.concatenate([jnp.zeros((hh, 8), f32), adst_bd], axis=1)

    S64 = jnp.concatenate([eye64, jnp.zeros((8, hh), f32)], axis=0)
    SA = jnp.concatenate([jnp.zeros((hh, heads), f32),
                          jnp.eye(heads, dtype=f32)], axis=0)
    E8 = jnp.repeat(jnp.eye(heads, dtype=f32), hid, axis=1)
    SD = jnp.concatenate([jnp.zeros((8, heads), f32),
                          jnp.eye(heads, dtype=f32)], axis=0)

    Pmu = jnp.zeros((z, W_CAT2), f32)
    Pmu = Pmu.at[:, 16:16 + z].set(eye16)
    Pmu = Pmu.at[:, 0].set(att_src_mu.reshape(z))
    Pstd = jnp.zeros((z, W_CAT2), f32)
    Pstd = Pstd.at[:, 16 + z:16 + 2 * z].set(eye16)
    Pstd = Pstd.at[:, 1].set(att_src_std.reshape(z))
    Qmu = jnp.zeros((z, W_DST2), f32).at[:, 0].set(att_dst_mu.reshape(z))
    Qstd = jnp.zeros((z, W_DST2), f32).at[:, 1].set(att_dst_std.reshape(z))

    Tmu = jnp.zeros((W_CAT2, z), f32).at[16:16 + z, :].set(eye16)
    Tstd = jnp.zeros((W_CAT2, z), f32).at[16 + z:16 + 2 * z, :].set(eye16)
    Bsmu = jnp.zeros((W_CAT2, z), f32).at[0, :].set(1.0)
    Bsstd = jnp.zeros((W_CAT2, z), f32).at[1, :].set(1.0)
    Umu = jnp.zeros((W_DST2, z), f32).at[0, :].set(1.0)
    Ustd = jnp.zeros((W_DST2, z), f32).at[1, :].set(1.0)

    chunk_e = 128
    rows_pad = -(-e_total // chunk_e)
    rows_pad = -(-rows_pad // (NC * NS * 8)) * (NC * NS * 8)
    e_pad = rows_pad * chunk_e
    cpt = rows_pad // (NC * NS)
    rpt = -(-(n + 1) // (NS * 8)) * 8
    n_pad = rpt * NS

    pad_idx = (n + jnp.arange(e_pad - e_total, dtype=edges.dtype)
               % jnp.asarray(n_pad - n, dtype=edges.dtype))
    src2d = jnp.concatenate([edges[0], pad_idx]).reshape(rows_pad, chunk_e)
    dst2d = jnp.concatenate([edges[1], pad_idx]).reshape(rows_pad, chunk_e)

    R = 2000
    grid = (n // R,)

    def _blk(w):
        return pl.BlockSpec((R, w), lambda i: (i, 0))

    def _full(a):
        return pl.BlockSpec(a.shape, lambda i: tuple(0 for _ in a.shape))

    cat1, adst1 = pl.pallas_call(
        _tc1_body,
        grid=grid,
        in_specs=[_blk(d_in), _full(W1), _full(P1), _full(D1)],
        out_specs=[_blk(W_CAT1), _blk(W_DST1)],
        out_shape=[jax.ShapeDtypeStruct((n, W_CAT1), f32),
                   jax.ShapeDtypeStruct((n, W_DST1), f32)],
    )(features, W1, P1, D1)
    cat1p = jnp.pad(cat1, ((0, n_pad - n), (0, 0)))
    adst1p = jnp.pad(adst1, ((0, n_pad - n), (0, 0)))

    mesh = plsc.VectorSubcoreMesh(core_axis_name="c", subcore_axis_name="s",
                                  num_cores=NC, num_subcores=NS)

    sc1 = pl.kernel(
        functools.partial(_sc1_body, cpt=cpt, rows_per_tile=rpt,
                          chunk_e=chunk_e),
        out_type=jax.ShapeDtypeStruct((NC, n_pad, W_CAT1), f32),
        mesh=mesh,
        compiler_params=pltpu.CompilerParams(use_tc_tiling_on_sc=False),
        scratch_types=[
            pltpu.VMEM((cpt, chunk_e), jnp.int32),
            pltpu.VMEM((cpt, chunk_e), jnp.int32),
            pltpu.VMEM((chunk_e, W_CAT1), f32),
            pltpu.VMEM((chunk_e, W_CAT1), f32),
            pltpu.VMEM((chunk_e, W_DST1), f32),
            pltpu.VMEM((chunk_e, W_DST1), f32),
            pltpu.VMEM((chunk_e, W_CAT1), f32),
            pltpu.VMEM((chunk_e, W_CAT1), f32),
            pltpu.VMEM_SHARED((n_pad, W_CAT1), f32),
            pltpu.SemaphoreType.DMA,
            pltpu.SemaphoreType.DMA,
            pltpu.SemaphoreType.DMA,
            pltpu.SemaphoreType.DMA,
            pltpu.SemaphoreType.DMA,
            pltpu.SemaphoreType.DMA,
        ],
    )
    acc1 = sc1(cat1p, adst1p, src2d, dst2d)

    b1r = b1.reshape(1, hh)
    cat2, adst2 = pl.pallas_call(
        _tc2_body,
        grid=grid,
        in_specs=[_blk(W_CAT1), _blk(W_CAT1), _blk(W_CAT1), _blk(W_DST1),
                  _full(b1r), _full(S64), _full(SA), _full(SD), _full(E8),
                  _full(W_mu), _full(W_std), _full(Pmu), _full(Pstd),
                  _full(Qmu), _full(Qstd)],
        out_specs=[_blk(W_CAT2), _blk(W_DST2)],
        out_shape=[jax.ShapeDtypeStruct((n, W_CAT2), f32),
                   jax.ShapeDtypeStruct((n, W_DST2), f32)],
    )(acc1[0, :n], acc1[1, :n], cat1, adst1, b1r,
      S64, SA, SD, E8, W_mu, W_std, Pmu, Pstd, Qmu, Qstd)

    cat2p = jnp.pad(cat2, ((0, n_pad - n), (0, 0)))
    adst2p = jnp.pad(adst2, ((0, n_pad - n), (0, 0)))
    sc2 = pl.kernel(
        functools.partial(_sc2_body, cpt=cpt, rows_per_tile=rpt,
                          chunk_e=chunk_e),
        out_type=jax.ShapeDtypeStruct((NC, n_pad, W_CAT2), f32),
        mesh=mesh,
        compiler_params=pltpu.CompilerParams(use_tc_tiling_on_sc=False),
        scratch_types=[
            pltpu.VMEM((cpt, chunk_e), jnp.int32),
            pltpu.VMEM((cpt, chunk_e), jnp.int32),
            pltpu.VMEM((chunk_e, W_CAT2), f32),
            pltpu.VMEM((chunk_e, W_CAT2), f32),
            pltpu.VMEM((chunk_e, W_DST2), f32),
            pltpu.VMEM((chunk_e, W_DST2), f32),
            pltpu.VMEM((chunk_e, W_CAT2), f32),
            pltpu.VMEM((chunk_e, W_CAT2), f32),
            pltpu.VMEM_SHARED((n_pad, W_CAT2), f32),
            pltpu.SemaphoreType.DMA,
            pltpu.SemaphoreType.DMA,
            pltpu.SemaphoreType.DMA,
            pltpu.SemaphoreType.DMA,
            pltpu.SemaphoreType.DMA,
            pltpu.SemaphoreType.DMA,
        ],
    )
    acc2 = sc2(cat2p, adst2p, src2d, dst2d)

    bmur = b_mu.reshape(1, z)
    bstdr = b_std.reshape(1, z)
    mu, std = pl.pallas_call(
        _tc3_body,
        grid=grid,
        in_specs=[_blk(W_CAT2), _blk(W_CAT2), _blk(W_CAT2), _blk(W_DST2),
                  _full(bmur), _full(bstdr), _full(Tmu), _full(Tstd),
                  _full(Bsmu), _full(Bsstd), _full(Umu), _full(Ustd)],
        out_specs=[_blk(z), _blk(z)],
        out_shape=[jax.ShapeDtypeStruct((n, z), f32),
                   jax.ShapeDtypeStruct((n, z), f32)],
    )(acc2[0, :n], acc2[1, :n], cat2, adst2, bmur, bstdr,
      Tmu, Tstd, Bsmu, Bsstd, Umu, Ustd)

    return (mu, std)
